# baseline (speedup 1.0000x reference)
import jax
import jax.numpy as jnp
from jax.experimental import pallas as pl
from jax.experimental.pallas import tpu as pltpu

B, N, CDIM = 8, 512, 3
K = 10
POOL = 4
FC1, FC2 = 512, 128

_LOG2E = 1.4426950408889634


def _stats_update(st_ref, h, b, k):
    dout = h.shape[-1]
    s1 = jnp.sum(h, axis=0, keepdims=True)
    s2 = jnp.sum(h * h, axis=0, keepdims=True)
    vals = jnp.stack([jnp.broadcast_to(s1, (K, dout)),
                      jnp.broadcast_to(s2, (K, dout))], axis=0)
    sel = jax.lax.broadcasted_iota(jnp.int32, (2, K, dout), 1) == k
    first = jnp.logical_and(b == 0, k == 0)
    prev = jnp.where(first, jnp.zeros_like(vals), st_ref[...])
    st_ref[...] = prev + jnp.where(sel, vals, 0.0)


def _matmul_tail(a_bf, vx_ref, w_ref, bb_ref, h_ref, st_ref, b, k):
    m = jnp.dot(a_bf, vx_ref[0].astype(jnp.bfloat16),
                preferred_element_type=jnp.float32)
    h = jnp.dot(m.astype(jnp.bfloat16), w_ref[0].astype(jnp.bfloat16),
                preferred_element_type=jnp.float32) + bb_ref[0]
    h_ref[0, 0] = h
    _stats_update(st_ref, h, b, k)


def _layer1_kernel(c_ref, ct_ref, vx_ref, w_ref, bb_ref, mu_ref, nv_ref,
                   h_ref, st_ref, a_ref, dm_ref):
    b = pl.program_id(0)
    k = pl.program_id(1)

    @pl.when(k == 0)
    def _():
        cb = c_ref[0]
        ct = ct_ref[0]
        acc = None
        for cc in range(CDIM):
            dif = cb[:, cc:cc + 1] - ct[cc:cc + 1, :]
            sq = dif * dif
            acc = sq if acc is None else acc + sq
        dm_ref[...] = jnp.sqrt(acc + 1e-12)

    mu_k = mu_ref[k]
    nv2_k = nv_ref[k]
    dm = dm_ref[...]
    d0 = dm - mu_k
    e = jax.lax.exp2(d0 * d0 * nv2_k)
    rs = jnp.sum(e, axis=1, keepdims=True)
    a = e * pl.reciprocal(rs + 1e-9, approx=True)
    a_bf = a.astype(jnp.bfloat16)
    a_ref[0, 0] = a_bf
    _matmul_tail(a_bf, vx_ref, w_ref, bb_ref, h_ref, st_ref, b, k)


def _layer1_call(C, CT, Vx, W, bb, mu, nv2):
    din = Vx.shape[-1]
    dout = W.shape[-1]
    return pl.pallas_call(
        _layer1_kernel,
        grid=(B, K),
        in_specs=[
            pl.BlockSpec((1, N, CDIM), lambda b, k: (b, 0, 0)),
            pl.BlockSpec((1, CDIM, N), lambda b, k: (b, 0, 0)),
            pl.BlockSpec((1, N, din), lambda b, k: (b, 0, 0)),
            pl.BlockSpec((1, din, dout), lambda b, k: (k, 0, 0)),
            pl.BlockSpec((1, 1, dout), lambda b, k: (k, 0, 0)),
            pl.BlockSpec(memory_space=pltpu.SMEM),
            pl.BlockSpec(memory_space=pltpu.SMEM),
        ],
        out_specs=[
            pl.BlockSpec((1, 1, N, dout), lambda b, k: (b, k, 0, 0)),
            pl.BlockSpec((2, K, dout), lambda b, k: (0, 0, 0)),
            pl.BlockSpec((1, 1, N, N), lambda b, k: (b, k, 0, 0)),
        ],
        out_shape=[
            jax.ShapeDtypeStruct((B, K, N, dout), jnp.float32),
            jax.ShapeDtypeStruct((2, K, dout), jnp.float32),
            jax.ShapeDtypeStruct((B, K, N, N), jnp.bfloat16),
        ],
        scratch_shapes=[pltpu.VMEM((N, N), jnp.float32)],
        compiler_params=pltpu.CompilerParams(
            dimension_semantics=("arbitrary", "arbitrary")),
    )(C, CT, Vx, W, bb, mu, nv2)


def _layerA_kernel(a_ref, vx_ref, w_ref, bb_ref, h_ref, st_ref):
    b = pl.program_id(0)
    k = pl.program_id(1)
    _matmul_tail(a_ref[0, 0], vx_ref, w_ref, bb_ref, h_ref, st_ref, b, k)


def _layerA_call(A, Vx, W, bb):
    din = Vx.shape[-1]
    dout = W.shape[-1]
    return pl.pallas_call(
        _layerA_kernel,
        grid=(B, K),
        in_specs=[
            pl.BlockSpec((1, 1, N, N), lambda b, k: (b, k, 0, 0)),
            pl.BlockSpec((1, N, din), lambda b, k: (b, 0, 0)),
            pl.BlockSpec((1, din, dout), lambda b, k: (k, 0, 0)),
            pl.BlockSpec((1, 1, dout), lambda b, k: (k, 0, 0)),
        ],
        out_specs=[
            pl.BlockSpec((1, 1, N, dout), lambda b, k: (b, k, 0, 0)),
            pl.BlockSpec((2, K, dout), lambda b, k: (0, 0, 0)),
        ],
        out_shape=[
            jax.ShapeDtypeStruct((B, K, N, dout), jnp.float32),
            jax.ShapeDtypeStruct((2, K, dout), jnp.float32),
        ],
        compiler_params=pltpu.CompilerParams(
            dimension_semantics=("arbitrary", "arbitrary")),
    )(A, Vx, W, bb)


def _bn_kernel(h_ref, st_ref, g_ref, be_ref, o_ref):
    inv_n = 1.0 / float(B * N)
    s1 = st_ref[0][:, None, :]
    s2 = st_ref[1][:, None, :]
    mean = s1 * inv_n
    var = s2 * inv_n - mean * mean
    rstd = jax.lax.rsqrt(var + 1e-5)
    scale = g_ref[...] * rstd
    shift = be_ref[...] - mean * scale
    h = h_ref[0]
    hn = h * scale + shift
    hs = hn * pl.reciprocal(1.0 + jnp.abs(hn), approx=True)
    o_ref[0] = jnp.mean(hs, axis=0)


def _bn_call(H, st, g, be):
    dout = H.shape[-1]
    return pl.pallas_call(
        _bn_kernel,
        grid=(B,),
        in_specs=[
            pl.BlockSpec((1, K, N, dout), lambda b: (b, 0, 0, 0)),
            pl.BlockSpec((2, K, dout), lambda b: (0, 0, 0)),
            pl.BlockSpec((K, 1, dout), lambda b: (0, 0, 0)),
            pl.BlockSpec((K, 1, dout), lambda b: (0, 0, 0)),
        ],
        out_specs=pl.BlockSpec((1, N, dout), lambda b: (b, 0, 0)),
        out_shape=jax.ShapeDtypeStruct((B, N, dout), jnp.float32),
        compiler_params=pltpu.CompilerParams(
            dimension_semantics=("arbitrary",)),
    )(H, st, g, be)


def _head_kernel(vx_ref, w1_ref, bf1_ref, w2_ref, bf2_ref, o_ref, acc_ref):
    j = pl.program_id(0)
    nj = pl.num_programs(0)
    d = vx_ref.shape[-1]
    rows = w1_ref.shape[0]

    @pl.when(j == 0)
    def _():
        acc_ref[...] = jnp.zeros_like(acc_ref)

    v = vx_ref[...]
    p = v.reshape(B, rows, POOL, d).max(axis=2)
    pb = p.astype(jnp.bfloat16)
    part = None
    for i in range(rows):
        t = jnp.dot(pb[:, i, :], w1_ref[i],
                    preferred_element_type=jnp.float32)
        part = t if part is None else part + t
    acc_ref[...] += part

    @pl.when(j == nj - 1)
    def _():
        h1 = jnp.maximum(acc_ref[...] + bf1_ref[...], 0.0)
        o = jnp.dot(h1.astype(jnp.bfloat16), w2_ref[...],
                    preferred_element_type=jnp.float32) + bf2_ref[...]
        o_ref[...] = jnp.maximum(o, 0.0)


def _head_call(Vx, W1r, bf1, W2, bf2):
    d = Vx.shape[-1]
    n2 = N // POOL
    rows = 16
    nsteps = n2 // rows
    return pl.pallas_call(
        _head_kernel,
        grid=(nsteps,),
        in_specs=[
            pl.BlockSpec((B, rows * POOL, d), lambda j: (0, j, 0)),
            pl.BlockSpec((rows, d, FC1), lambda j: (j, 0, 0)),
            pl.BlockSpec((1, FC1), lambda j: (0, 0)),
            pl.BlockSpec((FC1, FC2), lambda j: (0, 0)),
            pl.BlockSpec((1, FC2), lambda j: (0, 0)),
        ],
        out_specs=pl.BlockSpec((B, FC2), lambda j: (0, 0)),
        out_shape=jax.ShapeDtypeStruct((B, FC2), jnp.float32),
        scratch_shapes=[pltpu.VMEM((B, FC1), jnp.float32)],
        compiler_params=pltpu.CompilerParams(
            dimension_semantics=("arbitrary",)),
    )(Vx, W1r, bf1, W2, bf2)


def kernel(V, C, mu, sigma, W1, b1, g1, be1, W2, b2, g2, be2,
           W3, b3, g3, be3, Wf1, bf1, Wf2, bf2):
    CT = jnp.swapaxes(C, 1, 2)
    nv2 = -_LOG2E / (2.0 * sigma * sigma + 1e-6)

    H, st, A = _layer1_call(C, CT, V, W1, b1[:, None, :], mu, nv2)
    Vx = _bn_call(H, st, g1[:, None, :], be1[:, None, :])
    for W, bb, g, be in ((W2, b2, g2, be2), (W3, b3, g3, be3)):
        H, st = _layerA_call(A, Vx, W, bb[:, None, :])
        Vx = _bn_call(H, st, g[:, None, :], be[:, None, :])

    d = Vx.shape[-1]
    W1r = Wf1.reshape(N // POOL, d, FC1).astype(jnp.bfloat16)
    out = _head_call(Vx, W1r, bf1[None, :], Wf2.astype(jnp.bfloat16),
                     bf2[None, :])
    return out


# speedup vs baseline: 1.4383x; 1.4383x over previous
import jax
import jax.numpy as jnp
from jax.experimental import pallas as pl
from jax.experimental.pallas import tpu as pltpu

B, N, CDIM = 8, 512, 3
K = 10
POOL = 4
FC1, FC2 = 512, 128

_LOG2E = 1.4426950408889634


def _layer1_kernel(c_ref, ct_ref, vx_ref, w_ref, bb_ref, mu_ref, nv_ref,
                   a_ref, rv_ref, h_ref, st_ref, dm_ref):
    b = pl.program_id(0)
    k = pl.program_id(1)

    @pl.when(k == 0)
    def _():
        cb = c_ref[0]
        ct = ct_ref[0]
        acc = None
        for cc in range(CDIM):
            dif = cb[:, cc:cc + 1] - ct[cc:cc + 1, :]
            sq = dif * dif
            acc = sq if acc is None else acc + sq
        dm_ref[...] = jnp.sqrt(acc + 1e-12)

    mu_k = mu_ref[k]
    nv2_k = nv_ref[k]
    dm = dm_ref[...]
    d0 = dm - mu_k
    e = jax.lax.exp2(d0 * d0 * nv2_k)
    rs = jnp.sum(e, axis=1, keepdims=True)
    rinv = pl.reciprocal(rs + 1e-9, approx=True)
    e_bf = e.astype(jnp.bfloat16)
    a_ref[0] = e_bf
    rv_ref[0] = rinv
    m = jnp.dot(e_bf, vx_ref[0],
                preferred_element_type=jnp.float32) * rinv
    h = jnp.dot(m.astype(jnp.bfloat16), w_ref[0],
                preferred_element_type=jnp.float32) + bb_ref[0]
    h_ref[0] = h.astype(jnp.bfloat16)

    dout = h.shape[-1]
    s1 = jnp.sum(h, axis=0, keepdims=True)
    s2 = jnp.sum(h * h, axis=0, keepdims=True)
    vals = jnp.stack([jnp.broadcast_to(s1, (K, dout)),
                      jnp.broadcast_to(s2, (K, dout))], axis=0)
    sel = jax.lax.broadcasted_iota(jnp.int32, (2, K, dout), 1) == k
    first = jnp.logical_and(b == 0, k == 0)
    prev = jnp.where(first, jnp.zeros_like(vals), st_ref[...])
    st_ref[...] = prev + jnp.where(sel, vals, 0.0)


def _layer1_call(C, CT, Vb, W, bb, mu, nv2):
    din = Vb.shape[-1]
    dout = W.shape[-1]
    return pl.pallas_call(
        _layer1_kernel,
        grid=(B, K),
        in_specs=[
            pl.BlockSpec((1, N, CDIM), lambda b, k: (b, 0, 0)),
            pl.BlockSpec((1, CDIM, N), lambda b, k: (b, 0, 0)),
            pl.BlockSpec((1, N, din), lambda b, k: (b, 0, 0)),
            pl.BlockSpec((1, din, dout), lambda b, k: (k, 0, 0)),
            pl.BlockSpec((1, 1, dout), lambda b, k: (k, 0, 0)),
            pl.BlockSpec(memory_space=pltpu.SMEM),
            pl.BlockSpec(memory_space=pltpu.SMEM),
        ],
        out_specs=[
            pl.BlockSpec((1, N, N), lambda b, k: (b * K + k, 0, 0)),
            pl.BlockSpec((1, N, 1), lambda b, k: (b * K + k, 0, 0)),
            pl.BlockSpec((1, N, dout), lambda b, k: (b * K + k, 0, 0)),
            pl.BlockSpec((2, K, dout), lambda b, k: (0, 0, 0)),
        ],
        out_shape=[
            jax.ShapeDtypeStruct((B * K, N, N), jnp.bfloat16),
            jax.ShapeDtypeStruct((B * K, N, 1), jnp.float32),
            jax.ShapeDtypeStruct((B * K, N, dout), jnp.bfloat16),
            jax.ShapeDtypeStruct((2, K, dout), jnp.float32),
        ],
        scratch_shapes=[pltpu.VMEM((N, N), jnp.float32)],
        compiler_params=pltpu.CompilerParams(
            dimension_semantics=("arbitrary", "arbitrary")),
    )(C, CT, Vb, W, bb, mu, nv2)


def _layerA_kernel(a_ref, rv_ref, vx_ref, w_ref, bb_ref, h_ref, st_ref):
    b = pl.program_id(0)
    dout = w_ref.shape[-1]
    m_all = jnp.dot(a_ref[0], vx_ref[0],
                    preferred_element_type=jnp.float32)
    m_all = m_all * rv_ref[0]
    m_bf = m_all.astype(jnp.bfloat16)
    s1l, s2l = [], []
    for k in range(K):
        h = jnp.dot(m_bf[k * N:(k + 1) * N], w_ref[0, k],
                    preferred_element_type=jnp.float32) + bb_ref[0, k]
        h_ref[0, k * N:(k + 1) * N] = h.astype(jnp.bfloat16)
        s1l.append(jnp.sum(h, axis=0, keepdims=True))
        s2l.append(jnp.sum(h * h, axis=0, keepdims=True))
    s1 = jnp.concatenate(s1l, axis=0)
    s2 = jnp.concatenate(s2l, axis=0)
    vals = jnp.stack([s1, s2], axis=0)
    prev = jnp.where(b == 0, jnp.zeros_like(vals), st_ref[...])
    st_ref[...] = prev + vals


def _layerA_call(A, rv, Vx, W, bb):
    din = Vx.shape[-1]
    dout = W.shape[-1]
    return pl.pallas_call(
        _layerA_kernel,
        grid=(B,),
        in_specs=[
            pl.BlockSpec((1, K * N, N), lambda b: (b, 0, 0)),
            pl.BlockSpec((1, K * N, 1), lambda b: (b, 0, 0)),
            pl.BlockSpec((1, N, din), lambda b: (b, 0, 0)),
            pl.BlockSpec((1, K, din, dout), lambda b: (0, 0, 0, 0)),
            pl.BlockSpec((1, K, 1, dout), lambda b: (0, 0, 0, 0)),
        ],
        out_specs=[
            pl.BlockSpec((1, K * N, dout), lambda b: (b, 0, 0)),
            pl.BlockSpec((2, K, dout), lambda b: (0, 0, 0)),
        ],
        out_shape=[
            jax.ShapeDtypeStruct((B, K * N, dout), jnp.bfloat16),
            jax.ShapeDtypeStruct((2, K, dout), jnp.float32),
        ],
        compiler_params=pltpu.CompilerParams(
            dimension_semantics=("arbitrary",)),
    )(A, rv, Vx, W, bb)


def _bn_kernel(h_ref, st_ref, g_ref, be_ref, o_ref):
    inv_n = 1.0 / float(B * N)
    s1 = st_ref[0][:, None, :]
    s2 = st_ref[1][:, None, :]
    mean = s1 * inv_n
    var = s2 * inv_n - mean * mean
    rstd = jax.lax.rsqrt(var + 1e-5)
    scale = g_ref[...] * rstd
    shift = be_ref[...] - mean * scale
    h = h_ref[0].astype(jnp.float32)
    hn = h * scale + shift
    hs = hn * pl.reciprocal(1.0 + jnp.abs(hn), approx=True)
    o_ref[0] = jnp.mean(hs, axis=0).astype(jnp.bfloat16)


def _bn_call(H, st, g, be):
    dout = H.shape[-1]
    return pl.pallas_call(
        _bn_kernel,
        grid=(B,),
        in_specs=[
            pl.BlockSpec((1, K, N, dout), lambda b: (b, 0, 0, 0)),
            pl.BlockSpec((2, K, dout), lambda b: (0, 0, 0)),
            pl.BlockSpec((K, 1, dout), lambda b: (0, 0, 0)),
            pl.BlockSpec((K, 1, dout), lambda b: (0, 0, 0)),
        ],
        out_specs=pl.BlockSpec((1, N, dout), lambda b: (b, 0, 0)),
        out_shape=jax.ShapeDtypeStruct((B, N, dout), jnp.bfloat16),
        compiler_params=pltpu.CompilerParams(
            dimension_semantics=("arbitrary",)),
    )(H, st, g, be)


def _head_kernel(vx_ref, w1_ref, bf1_ref, w2_ref, bf2_ref, o_ref, acc_ref):
    j = pl.program_id(0)
    nj = pl.num_programs(0)
    d = vx_ref.shape[-1]
    rows = w1_ref.shape[0]

    @pl.when(j == 0)
    def _():
        acc_ref[...] = jnp.zeros_like(acc_ref)

    v = vx_ref[...]
    p = v.reshape(B, rows, POOL, d).max(axis=2)
    part = None
    for i in range(rows):
        t = jnp.dot(p[:, i, :], w1_ref[i],
                    preferred_element_type=jnp.float32)
        part = t if part is None else part + t
    acc_ref[...] += part

    @pl.when(j == nj - 1)
    def _():
        h1 = jnp.maximum(acc_ref[...] + bf1_ref[...], 0.0)
        o = jnp.dot(h1.astype(jnp.bfloat16), w2_ref[...],
                    preferred_element_type=jnp.float32) + bf2_ref[...]
        o_ref[...] = jnp.maximum(o, 0.0)


def _head_call(Vx, W1r, bf1, W2, bf2):
    d = Vx.shape[-1]
    n2 = N // POOL
    rows = 16
    nsteps = n2 // rows
    return pl.pallas_call(
        _head_kernel,
        grid=(nsteps,),
        in_specs=[
            pl.BlockSpec((B, rows * POOL, d), lambda j: (0, j, 0)),
            pl.BlockSpec((rows, d, FC1), lambda j: (j, 0, 0)),
            pl.BlockSpec((1, FC1), lambda j: (0, 0)),
            pl.BlockSpec((FC1, FC2), lambda j: (0, 0)),
            pl.BlockSpec((1, FC2), lambda j: (0, 0)),
        ],
        out_specs=pl.BlockSpec((B, FC2), lambda j: (0, 0)),
        out_shape=jax.ShapeDtypeStruct((B, FC2), jnp.float32),
        scratch_shapes=[pltpu.VMEM((B, FC1), jnp.float32)],
        compiler_params=pltpu.CompilerParams(
            dimension_semantics=("arbitrary",)),
    )(Vx, W1r, bf1, W2, bf2)


def kernel(V, C, mu, sigma, W1, b1, g1, be1, W2, b2, g2, be2,
           W3, b3, g3, be3, Wf1, bf1, Wf2, bf2):
    CT = jnp.swapaxes(C, 1, 2)
    nv2 = -_LOG2E / (2.0 * sigma * sigma + 1e-6)

    A, rv, H, st = _layer1_call(C, CT, V.astype(jnp.bfloat16),
                                W1.astype(jnp.bfloat16), b1[:, None, :],
                                mu, nv2)
    Vx = _bn_call(H.reshape(B, K, N, -1), st, g1[:, None, :],
                  be1[:, None, :])
    A = A.reshape(B, K * N, N)
    rv = rv.reshape(B, K * N, 1)
    for W, bb, g, be in ((W2, b2, g2, be2), (W3, b3, g3, be3)):
        H, st = _layerA_call(A, rv, Vx, W.astype(jnp.bfloat16)[None],
                             bb[None, :, None, :])
        Vx = _bn_call(H.reshape(B, K, N, -1), st, g[:, None, :],
                      be[:, None, :])

    d = Vx.shape[-1]
    W1r = Wf1.reshape(N // POOL, d, FC1).astype(jnp.bfloat16)
    out = _head_call(Vx, W1r, bf1[None, :], Wf2.astype(jnp.bfloat16),
                     bf2[None, :])
    return out
